# baseline (speedup 1.0000x reference)
import functools

import jax
import jax.numpy as jnp
from jax import lax
from jax.experimental import pallas as pl
from jax.experimental.pallas import tpu as pltpu
from jax.experimental.pallas import tpu_sc as plsc

NC = 2
NS = 16
LANES = 16
NW = NC * NS

EC = 16
IDX_CHUNK = 128


def _sc_body(pw, neg, emb, ixr_hbm, ixh_hbm, iar_hbm, iah_hbm, w_hid2,
             w_out2, pos_out, negs_out,
             xi_v, xh_v, ia_v, ih_v, xb0, xb1, rows0, rows1,
             pos_v, negs_v, sem0, sem1):
    npe = neg + 1
    nseg = emb // LANES
    steps = pw // EC
    rows_n = EC * npe
    chunks = [(o, min(IDX_CHUNK, rows_n - o)) for o in range(0, rows_n, IDX_CHUNK)]
    wid = lax.axis_index("s") * NC + lax.axis_index("c")
    base = wid * pw
    lanes = lax.iota(jnp.int32, 16)
    lane0 = lanes == 0

    def scatter1(ref, pos_i, val):
        plsc.store_scatter(ref, [jnp.broadcast_to(pos_i, (16,))],
                           jnp.broadcast_to(val, (16,)), mask=lane0)

    def splat_ld(ref, i):
        return plsc.load_gather(ref, [jnp.broadcast_to(i, (16,))])

    pltpu.sync_copy(ixr_hbm.at[pl.ds(base, pw)], xi_v)
    pltpu.sync_copy(ixh_hbm.at[pl.ds(base, pw)], xh_v)
    pltpu.sync_copy(iar_hbm.at[pl.ds(base * npe, pw * npe)], ia_v)
    pltpu.sync_copy(iah_hbm.at[pl.ds(base * npe, pw * npe)], ih_v)

    def fire(s, xbuf, rbuf, sem):
        pltpu.async_copy(w_hid2.at[xi_v.at[pl.ds(s * EC, EC)]], xbuf, sem)
        off = s * rows_n
        for o, c in chunks:
            pltpu.async_copy(w_out2.at[ia_v.at[pl.ds(off + o, c)]],
                             rbuf.at[pl.ds(o, c)], sem)

    def drain(xbuf, rbuf, sem):
        pltpu.make_async_copy(
            w_hid2.at[xi_v.at[pl.ds(0, EC)]], xbuf, sem).wait()
        for o, c in chunks:
            pltpu.make_async_copy(w_out2.at[ia_v.at[pl.ds(o, c)]],
                                  rbuf.at[pl.ds(o, c)], sem).wait()

    fire(0, xb0, rows0, sem0)

    def compute(s, xbuf, rbuf):
        def elem(e, carry):
            b = s * EC + e
            xcol = lanes + splat_ld(xh_v, b)
            erow = jnp.broadcast_to(e, (16,))
            xr = [plsc.load_gather(xbuf, [erow, xcol + k * LANES])
                  for k in range(nseg)]

            def dot_row(j):
                rrow = jnp.broadcast_to(e * npe + j, (16,))
                col = lanes + splat_ld(ih_v, b * npe + j)
                acc = plsc.load_gather(rbuf, [rrow, col]) * xr[0]
                for k in range(1, nseg):
                    acc = acc + plsc.load_gather(
                        rbuf, [rrow, col + k * LANES]) * xr[k]
                return jnp.sum(acc)

            scatter1(pos_v, b, dot_row(0))
            for n_i in range(neg):
                scatter1(negs_v, b * neg + n_i, dot_row(1 + n_i))
            return carry

        lax.fori_loop(0, EC, elem, 0)

    def outer(t, carry):
        s0 = 2 * t
        fire(s0 + 1, xb1, rows1, sem1)
        drain(xb0, rows0, sem0)
        compute(s0, xb0, rows0)
        fire(jnp.minimum(s0 + 2, steps - 1), xb0, rows0, sem0)
        drain(xb1, rows1, sem1)
        compute(s0 + 1, xb1, rows1)
        return carry

    lax.fori_loop(0, steps // 2, outer, 0)
    drain(xb0, rows0, sem0)

    pltpu.sync_copy(pos_v, pos_out.at[pl.ds(base, pw)])
    pltpu.sync_copy(negs_v, negs_out.at[pl.ds(base * neg, pw * neg)])


def _tc_loss_body(pos_ref, neg_ref, out_ref):
    p = pos_ref[...]
    n = neg_ref[...]
    ls_p = jnp.minimum(p, 0.0) - jnp.log1p(jnp.exp(-jnp.abs(p)))
    ls_n = jnp.minimum(-n, 0.0) - jnp.log1p(jnp.exp(-jnp.abs(n)))
    out_ref[0, 0] = -(jnp.sum(ls_p) + jnp.sum(ls_n))


def kernel(positive_pairs, negative_samples, W_hid, W_out):
    batch, neg = negative_samples.shape
    vocab, emb = W_hid.shape
    pw = batch // NW
    npe = neg + 1

    ix = positive_pairs[:, 0]
    ia = jnp.concatenate(
        [positive_pairs[:, 1:2], negative_samples], axis=1).reshape(-1)
    w_hid2 = W_hid.reshape(vocab // 2, 2 * emb)
    w_out2 = W_out.reshape(vocab // 2, 2 * emb)
    ixr, ixh = ix >> 1, (ix & 1) * emb
    iar, iah = ia >> 1, (ia & 1) * emb

    mesh = plsc.VectorSubcoreMesh(
        core_axis_name="c", subcore_axis_name="s",
        num_cores=NC, num_subcores=NS)
    sc_scores = pl.kernel(
        functools.partial(_sc_body, pw, neg, emb),
        out_type=(jax.ShapeDtypeStruct((batch,), jnp.float32),
                  jax.ShapeDtypeStruct((batch * neg,), jnp.float32)),
        mesh=mesh,
        scratch_types=[
            pltpu.VMEM((pw,), jnp.int32),
            pltpu.VMEM((pw,), jnp.int32),
            pltpu.VMEM((pw * npe,), jnp.int32),
            pltpu.VMEM((pw * npe,), jnp.int32),
            pltpu.VMEM((EC, 2 * emb), jnp.float32),
            pltpu.VMEM((EC, 2 * emb), jnp.float32),
            pltpu.VMEM((EC * npe, 2 * emb), jnp.float32),
            pltpu.VMEM((EC * npe, 2 * emb), jnp.float32),
            pltpu.VMEM((pw,), jnp.float32),
            pltpu.VMEM((pw * neg,), jnp.float32),
            pltpu.SemaphoreType.DMA,
            pltpu.SemaphoreType.DMA,
        ],
        compiler_params=pltpu.CompilerParams(
            needs_layout_passes=False, use_tc_tiling_on_sc=False),
    )
    pos_s, neg_s = sc_scores(ixr, ixh, iar, iah, w_hid2, w_out2)

    pos2 = pos_s.reshape(batch // 128, 128)
    neg2 = neg_s.reshape(batch * neg // 128, 128)
    loss = pl.pallas_call(
        _tc_loss_body,
        out_shape=jax.ShapeDtypeStruct((1, 1), jnp.float32),
        out_specs=pl.BlockSpec(memory_space=pltpu.SMEM),
    )(pos2, neg2)
    return loss[0, 0]


# speedup vs baseline: 1.0709x; 1.0709x over previous
import functools

import jax
import jax.numpy as jnp
from jax import lax
from jax.experimental import pallas as pl
from jax.experimental.pallas import tpu as pltpu
from jax.experimental.pallas import tpu_sc as plsc

NC = 2
NS = 16
LANES = 16
NW = NC * NS

EC = 16
IDX_CHUNK = 128


def _sc_body(pw, neg, emb, ix_hbm, ia_hbm, w_hid, w_out, pos_out, negs_out,
             xi_v, ia_v, x_v, rows0, rows1, pos_v, negs_v, semx, sem0, sem1):
    npe = neg + 1
    nseg = emb // LANES
    steps = pw // EC
    rows_n = EC * npe
    chunks = [(o, min(IDX_CHUNK, rows_n - o)) for o in range(0, rows_n, IDX_CHUNK)]
    wid = lax.axis_index("s") * NC + lax.axis_index("c")
    base = wid * pw
    lane0 = lax.iota(jnp.int32, 16) == 0

    def scatter1(ref, pos_i, val):
        plsc.store_scatter(ref, [jnp.broadcast_to(pos_i, (16,))],
                           jnp.broadcast_to(val, (16,)), mask=lane0)

    pltpu.sync_copy(ix_hbm.at[pl.ds(base, pw)], xi_v)
    pltpu.sync_copy(ia_hbm.at[pl.ds(base * npe, pw * npe)], ia_v)

    xcps = [pltpu.make_async_copy(
        w_hid.at[xi_v.at[pl.ds(j * IDX_CHUNK, IDX_CHUNK)]],
        x_v.at[pl.ds(j * IDX_CHUNK, IDX_CHUNK)], semx)
        for j in range(pw // IDX_CHUNK)]
    for cp in xcps:
        cp.start()

    def fire(s, rbuf, sem):
        off = s * rows_n
        for o, c in chunks:
            pltpu.async_copy(w_out.at[ia_v.at[pl.ds(off + o, c)]],
                             rbuf.at[pl.ds(o, c)], sem)

    def drain(rbuf, sem):
        for o, c in chunks:
            pltpu.make_async_copy(w_out.at[ia_v.at[pl.ds(o, c)]],
                                  rbuf.at[pl.ds(o, c)], sem).wait()

    fire(0, rows0, sem0)
    for cp in xcps:
        cp.wait()

    def compute(s, rbuf):
        def elem(e, carry):
            b = s * EC + e
            xr = [x_v[b, pl.ds(k * LANES, LANES)] for k in range(nseg)]
            r0 = e * npe
            yr = [rbuf[r0, pl.ds(k * LANES, LANES)] for k in range(nseg)]
            acc = xr[0] * yr[0]
            for k in range(1, nseg):
                acc = acc + xr[k] * yr[k]
            scatter1(pos_v, b, jnp.sum(acc))
            for n_i in range(neg):
                nr = [rbuf[r0 + 1 + n_i, pl.ds(k * LANES, LANES)]
                      for k in range(nseg)]
                nacc = nr[0] * xr[0]
                for k in range(1, nseg):
                    nacc = nacc + nr[k] * xr[k]
                scatter1(negs_v, b * neg + n_i, jnp.sum(nacc))
            return carry

        lax.fori_loop(0, EC, elem, 0)

    def outer(t, carry):
        s0 = 2 * t
        fire(s0 + 1, rows1, sem1)
        drain(rows0, sem0)
        compute(s0, rows0)
        fire(jnp.minimum(s0 + 2, steps - 1), rows0, sem0)
        drain(rows1, sem1)
        compute(s0 + 1, rows1)
        return carry

    lax.fori_loop(0, steps // 2, outer, 0)
    drain(rows0, sem0)

    pltpu.sync_copy(pos_v, pos_out.at[pl.ds(base, pw)])
    pltpu.sync_copy(negs_v, negs_out.at[pl.ds(base * neg, pw * neg)])


def _tc_loss_body(pos_ref, neg_ref, out_ref):
    p = pos_ref[...]
    n = neg_ref[...]
    ls_p = jnp.minimum(p, 0.0) - jnp.log1p(jnp.exp(-jnp.abs(p)))
    ls_n = jnp.minimum(-n, 0.0) - jnp.log1p(jnp.exp(-jnp.abs(n)))
    out_ref[0, 0] = -(jnp.sum(ls_p) + jnp.sum(ls_n))


def kernel(positive_pairs, negative_samples, W_hid, W_out):
    batch, neg = negative_samples.shape
    emb = W_hid.shape[1]
    pw = batch // NW
    npe = neg + 1

    ix = positive_pairs[:, 0]
    ia = jnp.concatenate(
        [positive_pairs[:, 1:2], negative_samples], axis=1).reshape(-1)

    mesh = plsc.VectorSubcoreMesh(
        core_axis_name="c", subcore_axis_name="s",
        num_cores=NC, num_subcores=NS)
    sc_scores = pl.kernel(
        functools.partial(_sc_body, pw, neg, emb),
        out_type=(jax.ShapeDtypeStruct((batch,), jnp.float32),
                  jax.ShapeDtypeStruct((batch * neg,), jnp.float32)),
        mesh=mesh,
        scratch_types=[
            pltpu.VMEM((pw,), jnp.int32),
            pltpu.VMEM((pw * npe,), jnp.int32),
            pltpu.VMEM((pw, emb), jnp.float32),
            pltpu.VMEM((EC * npe, emb), jnp.float32),
            pltpu.VMEM((EC * npe, emb), jnp.float32),
            pltpu.VMEM((pw,), jnp.float32),
            pltpu.VMEM((pw * neg,), jnp.float32),
            pltpu.SemaphoreType.DMA,
            pltpu.SemaphoreType.DMA,
            pltpu.SemaphoreType.DMA,
        ],
        compiler_params=pltpu.CompilerParams(
            needs_layout_passes=False, use_tc_tiling_on_sc=False),
    )
    pos_s, neg_s = sc_scores(ix, ia, W_hid, W_out)

    pos2 = pos_s.reshape(batch // 128, 128)
    neg2 = neg_s.reshape(batch * neg // 128, 128)
    loss = pl.pallas_call(
        _tc_loss_body,
        out_shape=jax.ShapeDtypeStruct((1, 1), jnp.float32),
        out_specs=pl.BlockSpec(memory_space=pltpu.SMEM),
    )(pos2, neg2)
    return loss[0, 0]
